# baseline (speedup 1.0000x reference)
import functools

import jax
import jax.numpy as jnp
from jax import lax
from jax.experimental import pallas as pl
from jax.experimental.pallas import tpu as pltpu
from jax.experimental.pallas import tpu_sc as plsc

NC = 2
NS = 16
NW = NC * NS

D = 64
L = 200
B = 4096
TOTAL = B * L
CHUNK = 128
PER_W = TOTAL // NW
N_CHUNKS = PER_W // CHUNK
NBUF = 3
POS_EXT = L + CHUNK


def _body(idx_hbm, tok_hbm, pos_hbm, out_hbm, idx_v, pos_v,
          b0, b1, b2, g0, g1, g2, s0, s1, s2):
  bufs = (b0, b1, b2)
  gsems = (g0, g1, g2)
  ssems = (s0, s1, s2)

  wid = lax.axis_index("s") * NC + lax.axis_index("c")
  row0 = wid * PER_W
  crow0 = wid * N_CHUNKS

  pltpu.sync_copy(idx_hbm.at[pl.ds(crow0, N_CHUNKS)], idx_v)
  pltpu.sync_copy(pos_hbm, pos_v.at[pl.ds(0, L)])
  pltpu.sync_copy(pos_hbm.at[pl.ds(0, CHUNK)], pos_v.at[pl.ds(L, CHUNK)])

  def start_gather(k, p):
    pltpu.async_copy(tok_hbm.at[idx_v.at[k]], bufs[p], gsems[p])

  def wait_gather(k, p):
    pltpu.make_async_copy(tok_hbm.at[idx_v.at[k]], bufs[p], gsems[p]).wait()

  def start_store(k, p):
    dst = out_hbm.at[pl.ds(row0 + k * CHUNK, CHUNK)]
    pltpu.async_copy(bufs[p], dst, ssems[p])

  def wait_store(k, p):
    dst = out_hbm.at[pl.ds(row0 + k * CHUNK, CHUNK)]
    pltpu.make_async_copy(bufs[p], dst, ssems[p]).wait()

  def add_pos(k, p):
    buf = bufs[p]
    phase = lax.rem(k * CHUNK, L)

    def rows(j, carry):
      r = j * 4
      for u in range(4):
        rr = r + u
        pr = phase + rr
        for v in range(D // 16):
          s = pl.ds(v * 16, 16)
          buf[rr, s] = buf[rr, s] + pos_v[pr, s]
      return carry

    lax.fori_loop(0, CHUNK // 4, rows, 0)

  start_gather(0, 0)
  start_gather(1, 1)

  start_gather(2, 2)
  wait_gather(0, 0)
  add_pos(0, 0)
  start_store(0, 0)

  def group(g, carry):
    for p_off in range(NBUF):
      k = NBUF * g + 1 + p_off
      pc = (1 + p_off) % NBUF
      pn = (p_off) % NBUF
      wait_store(k - 1, pn)
      start_gather(k + 2, pn)
      wait_gather(k, pc)
      add_pos(k, pc)
      start_store(k, pc)
    return carry

  lax.fori_loop(0, (N_CHUNKS - 5) // NBUF, group, 0)

  for k in range(N_CHUNKS - 4, N_CHUNKS):
    pc = k % NBUF
    pn = (k - 1) % NBUF
    wait_store(k - 1, pn)
    if k + 2 < N_CHUNKS:
      start_gather(k + 2, pn)
    wait_gather(k, pc)
    add_pos(k, pc)
    start_store(k, pc)
  wait_store(N_CHUNKS - 1, (N_CHUNKS - 1) % NBUF)


@jax.jit
def _run(idx_flat, token_table, position_table):
  mesh = plsc.VectorSubcoreMesh(
      core_axis_name="c", subcore_axis_name="s", num_cores=NC,
      num_subcores=NS)
  f = pl.kernel(
      _body,
      out_type=jax.ShapeDtypeStruct((TOTAL, D), jnp.float32),
      mesh=mesh,
      scratch_types=[
          pltpu.VMEM((N_CHUNKS, CHUNK), jnp.int32),
          pltpu.VMEM((POS_EXT, D), jnp.float32),
          pltpu.VMEM((CHUNK, D), jnp.float32),
          pltpu.VMEM((CHUNK, D), jnp.float32),
          pltpu.VMEM((CHUNK, D), jnp.float32),
          pltpu.SemaphoreType.DMA,
          pltpu.SemaphoreType.DMA,
          pltpu.SemaphoreType.DMA,
          pltpu.SemaphoreType.DMA,
          pltpu.SemaphoreType.DMA,
          pltpu.SemaphoreType.DMA,
      ],
      compiler_params=pltpu.CompilerParams(use_tc_tiling_on_sc=False),
  )
  return f(idx_flat, token_table, position_table)


def kernel(inputs, token_table, position_table):
  idx = inputs.astype(jnp.int32).reshape(TOTAL // CHUNK, CHUNK)
  out = _run(idx, token_table, position_table)
  return out.reshape(B, L, D)


# speedup vs baseline: 1.1487x; 1.1487x over previous
import functools

import jax
import jax.numpy as jnp
from jax import lax
from jax.experimental import pallas as pl
from jax.experimental.pallas import tpu as pltpu
from jax.experimental.pallas import tpu_sc as plsc

NC = 2
NS = 16
NW = NC * NS

D = 64
L = 200
B = 4096
TOTAL = B * L
CHUNK = 128
PER_W = TOTAL // NW
N_CHUNKS = PER_W // CHUNK
NBUF = 3
HL = L // 2
POS_EXT = HL + CHUNK // 2
VOCAB_HALF = 500000


def _body(idx_hbm, tok_hbm, pos_hbm, out_hbm, idx_v, pos_v,
          b0, b1, b2, t0, t1, t2, h0, h1, h2, g0, g1, g2, s0, s1, s2):
  bufs = (b0, b1, b2)
  stages = (t0, t1, t2)
  idxhs = (h0, h1, h2)
  gsems = (g0, g1, g2)
  ssems = (s0, s1, s2)

  wid = lax.axis_index("s") * NC + lax.axis_index("c")
  crow0 = wid * N_CHUNKS
  orow0 = wid * (PER_W // 2)

  pltpu.sync_copy(idx_hbm.at[pl.ds(crow0, N_CHUNKS)], idx_v)
  pltpu.sync_copy(pos_hbm, pos_v.at[pl.ds(0, HL)])
  pltpu.sync_copy(pos_hbm.at[pl.ds(0, POS_EXT - HL)],
                  pos_v.at[pl.ds(HL, POS_EXT - HL)])

  def start_gather(k, p):
    for g in range(CHUNK // 16):
      s = pl.ds(g * 16, 16)
      idxhs[p][s] = lax.shift_right_logical(idx_v[k, s], 1)
    pltpu.async_copy(tok_hbm.at[idxhs[p]], bufs[p], gsems[p])

  def wait_gather(p):
    pltpu.make_async_copy(tok_hbm.at[idxhs[p]], bufs[p], gsems[p]).wait()

  def start_store(k, p):
    dst = out_hbm.at[pl.ds(orow0 + k * (CHUNK // 2), CHUNK // 2)]
    pltpu.async_copy(stages[p], dst, ssems[p])

  def wait_store(k, p):
    dst = out_hbm.at[pl.ds(orow0 + k * (CHUNK // 2), CHUNK // 2)]
    pltpu.make_async_copy(stages[p], dst, ssems[p]).wait()

  def add_pos(k, p):
    buf = bufs[p]
    stage = stages[p]
    phase2 = lax.rem(k * (CHUNK // 2), HL)

    @plsc.parallel_loop(0, CHUNK // 16, unroll=2)
    def rows(g):
      pv = (idx_v[k, pl.ds(g * 16, 16)] & 1) * D
      for u in range(16):
        j = g * 8 + u // 2
        pr = phase2 + j
        off = pv[u]
        for v in range(D // 16):
          so = pl.ds((u % 2) * D + v * 16, 16)
          stage[j, so] = buf[g * 16 + u, pl.ds(off + v * 16, 16)] + pos_v[pr, so]

  start_gather(0, 0)
  start_gather(1, 1)

  start_gather(2, 2)
  wait_gather(0)
  add_pos(0, 0)
  start_store(0, 0)

  def group(g, carry):
    for p_off in range(NBUF):
      k = NBUF * g + 1 + p_off
      pc = (1 + p_off) % NBUF
      pn = (p_off) % NBUF
      wait_store(k - 1, pn)
      start_gather(k + 2, pn)
      wait_gather(pc)
      add_pos(k, pc)
      start_store(k, pc)
    return carry

  lax.fori_loop(0, (N_CHUNKS - 5) // NBUF, group, 0)

  for k in range(N_CHUNKS - 4, N_CHUNKS):
    pc = k % NBUF
    pn = (k - 1) % NBUF
    wait_store(k - 1, pn)
    if k + 2 < N_CHUNKS:
      start_gather(k + 2, pn)
    wait_gather(pc)
    add_pos(k, pc)
    start_store(k, pc)
  wait_store(N_CHUNKS - 1, (N_CHUNKS - 1) % NBUF)


@jax.jit
def _run(idx, tok2, pos2):
  mesh = plsc.VectorSubcoreMesh(
      core_axis_name="c", subcore_axis_name="s", num_cores=NC,
      num_subcores=NS)
  f = pl.kernel(
      _body,
      out_type=jax.ShapeDtypeStruct((TOTAL // 2, 2 * D), jnp.float32),
      mesh=mesh,
      scratch_types=[
          pltpu.VMEM((N_CHUNKS, CHUNK), jnp.int32),
          pltpu.VMEM((POS_EXT, 2 * D), jnp.float32),
          pltpu.VMEM((CHUNK, 2 * D), jnp.float32),
          pltpu.VMEM((CHUNK, 2 * D), jnp.float32),
          pltpu.VMEM((CHUNK, 2 * D), jnp.float32),
          pltpu.VMEM((CHUNK // 2, 2 * D), jnp.float32),
          pltpu.VMEM((CHUNK // 2, 2 * D), jnp.float32),
          pltpu.VMEM((CHUNK // 2, 2 * D), jnp.float32),
          pltpu.VMEM((CHUNK,), jnp.int32),
          pltpu.VMEM((CHUNK,), jnp.int32),
          pltpu.VMEM((CHUNK,), jnp.int32),
          pltpu.SemaphoreType.DMA,
          pltpu.SemaphoreType.DMA,
          pltpu.SemaphoreType.DMA,
          pltpu.SemaphoreType.DMA,
          pltpu.SemaphoreType.DMA,
          pltpu.SemaphoreType.DMA,
      ],
  )
  return f(idx, tok2, pos2)


def kernel(inputs, token_table, position_table):
  idx = inputs.astype(jnp.int32).reshape(TOTAL // CHUNK, CHUNK)
  tok2 = token_table.reshape(VOCAB_HALF, 2 * D)
  pos2 = position_table.reshape(HL, 2 * D)
  out2 = _run(idx, tok2, pos2)
  return out2.reshape(B, L, D)


# speedup vs baseline: 1.2146x; 1.0574x over previous
import functools

import jax
import jax.numpy as jnp
from jax import lax
from jax.experimental import pallas as pl
from jax.experimental.pallas import tpu as pltpu
from jax.experimental.pallas import tpu_sc as plsc

NC = 2
NS = 16
NW = NC * NS

D = 64
L = 200
B = 4096
TOTAL = B * L
CHUNK = 128
PER_W = TOTAL // NW
N_CHUNKS = PER_W // CHUNK
NBUF = 3
HL = L // 2
POS_EXT = HL + CHUNK // 2
VOCAB_HALF = 500000


def _body(idx_hbm, tok_hbm, pos_hbm, out_hbm, idx_v, pos_v,
          b0, b1, b2, t0, t1, t2, h0, h1, h2, g0, g1, g2, s0, s1, s2):
  bufs = (b0, b1, b2)
  stages = (t0, t1, t2)
  idxhs = (h0, h1, h2)
  gsems = (g0, g1, g2)
  ssems = (s0, s1, s2)

  wid = lax.axis_index("s") * NC + lax.axis_index("c")
  crow0 = wid * N_CHUNKS
  orow0 = wid * (PER_W // 2)

  pltpu.sync_copy(idx_hbm.at[pl.ds(crow0, N_CHUNKS)], idx_v)
  pltpu.sync_copy(pos_hbm, pos_v.at[pl.ds(0, HL)])
  pltpu.sync_copy(pos_hbm.at[pl.ds(0, POS_EXT - HL)],
                  pos_v.at[pl.ds(HL, POS_EXT - HL)])

  def start_gather(k, p):
    for g in range(CHUNK // 16):
      s = pl.ds(g * 16, 16)
      idxhs[p][s] = idx_v[k, s]
    pltpu.async_copy(tok_hbm.at[idxhs[p]], bufs[p], gsems[p])

  def wait_gather(p):
    pltpu.make_async_copy(tok_hbm.at[idxhs[p]], bufs[p], gsems[p]).wait()

  def start_store(k, p):
    dst = out_hbm.at[pl.ds(orow0 + k * (CHUNK // 2), CHUNK // 2)]
    pltpu.async_copy(stages[p], dst, ssems[p])

  def wait_store(k, p):
    dst = out_hbm.at[pl.ds(orow0 + k * (CHUNK // 2), CHUNK // 2)]
    pltpu.make_async_copy(stages[p], dst, ssems[p]).wait()

  def add_pos(k, p):
    buf = bufs[p]
    stage = stages[p]
    phase2 = lax.rem(k * (CHUNK // 2), HL)

    @plsc.parallel_loop(0, CHUNK // 16, unroll=2)
    def rows(g):
      for u in range(16):
        j = g * 8 + u // 2
        pr = phase2 + j
        for v in range(D // 16):
          so = pl.ds((u % 2) * D + v * 16, 16)
          stage[j, so] = buf[g * 16 + u, pl.ds(v * 16, 16)] + pos_v[pr, so]

  start_gather(0, 0)
  start_gather(1, 1)

  start_gather(2, 2)
  wait_gather(0)
  add_pos(0, 0)
  start_store(0, 0)

  def group(g, carry):
    for p_off in range(NBUF):
      k = NBUF * g + 1 + p_off
      pc = (1 + p_off) % NBUF
      pn = (p_off) % NBUF
      wait_store(k - 1, pn)
      start_gather(k + 2, pn)
      wait_gather(pc)
      add_pos(k, pc)
      start_store(k, pc)
    return carry

  lax.fori_loop(0, (N_CHUNKS - 5) // NBUF, group, 0)

  for k in range(N_CHUNKS - 4, N_CHUNKS):
    pc = k % NBUF
    pn = (k - 1) % NBUF
    wait_store(k - 1, pn)
    if k + 2 < N_CHUNKS:
      start_gather(k + 2, pn)
    wait_gather(pc)
    add_pos(k, pc)
    start_store(k, pc)
  wait_store(N_CHUNKS - 1, (N_CHUNKS - 1) % NBUF)


@jax.jit
def _run(idx, tok2, pos2):
  mesh = plsc.VectorSubcoreMesh(
      core_axis_name="c", subcore_axis_name="s", num_cores=NC,
      num_subcores=NS)
  f = pl.kernel(
      _body,
      out_type=jax.ShapeDtypeStruct((TOTAL // 2, 2 * D), jnp.float32),
      mesh=mesh,
      scratch_types=[
          pltpu.VMEM((N_CHUNKS, CHUNK), jnp.int32),
          pltpu.VMEM((POS_EXT, 2 * D), jnp.float32),
          pltpu.VMEM((CHUNK, 2 * D), jnp.float32),
          pltpu.VMEM((CHUNK, 2 * D), jnp.float32),
          pltpu.VMEM((CHUNK, 2 * D), jnp.float32),
          pltpu.VMEM((CHUNK // 2, 2 * D), jnp.float32),
          pltpu.VMEM((CHUNK // 2, 2 * D), jnp.float32),
          pltpu.VMEM((CHUNK // 2, 2 * D), jnp.float32),
          pltpu.VMEM((CHUNK,), jnp.int32),
          pltpu.VMEM((CHUNK,), jnp.int32),
          pltpu.VMEM((CHUNK,), jnp.int32),
          pltpu.SemaphoreType.DMA,
          pltpu.SemaphoreType.DMA,
          pltpu.SemaphoreType.DMA,
          pltpu.SemaphoreType.DMA,
          pltpu.SemaphoreType.DMA,
          pltpu.SemaphoreType.DMA,
      ],
  )
  return f(idx, tok2, pos2)


def kernel(inputs, token_table, position_table):
  idx = inputs.astype(jnp.int32).reshape(TOTAL // CHUNK, CHUNK)
  tok2 = jnp.pad(token_table, ((0, 0), (0, D)))
  pos2 = position_table.reshape(HL, 2 * D)
  out2 = _run(idx, tok2, pos2)
  return out2.reshape(B, L, D)


# speedup vs baseline: 1.4667x; 1.2075x over previous
import functools

import jax
import jax.numpy as jnp
from jax import lax
from jax.experimental import pallas as pl
from jax.experimental.pallas import tpu as pltpu
from jax.experimental.pallas import tpu_sc as plsc

NC = 2
NS = 16
NW = NC * NS

D = 64
L = 200
B = 4096
TOTAL = B * L
CHUNK = 128
PER_W = TOTAL // NW
N_CHUNKS = PER_W // CHUNK
NBUF = 3
HL = L // 2
POS_EXT = HL + CHUNK // 2


def _body(idx_hbm, tok_hbm, pos_hbm, out_hbm, idx_v, pos_v,
          b0, b1, b2, h0, h1, h2, g0, g1, g2, s0, s1, s2):
  bufs = (b0, b1, b2)
  idxcs = (h0, h1, h2)
  gsems = (g0, g1, g2)
  ssems = (s0, s1, s2)

  wid = lax.axis_index("s") * NC + lax.axis_index("c")
  crow0 = wid * N_CHUNKS
  row0 = wid * PER_W

  pltpu.sync_copy(idx_hbm.at[pl.ds(crow0, N_CHUNKS)], idx_v)
  pltpu.sync_copy(pos_hbm, pos_v.at[pl.ds(0, HL)])
  pltpu.sync_copy(pos_hbm.at[pl.ds(0, POS_EXT - HL)],
                  pos_v.at[pl.ds(HL, POS_EXT - HL)])

  def start_gather(k, p):
    for g in range(CHUNK // 16):
      s = pl.ds(g * 16, 16)
      idxcs[p][s] = idx_v[k, s]
    pltpu.async_copy(tok_hbm.at[idxcs[p]], bufs[p], gsems[p])

  def wait_gather(p):
    pltpu.make_async_copy(tok_hbm.at[idxcs[p]], bufs[p], gsems[p]).wait()

  def start_store(k, p):
    dst = out_hbm.at[pl.ds(row0 + k * CHUNK, CHUNK)]
    pltpu.async_copy(bufs[p], dst, ssems[p])

  def wait_store(k, p):
    dst = out_hbm.at[pl.ds(row0 + k * CHUNK, CHUNK)]
    pltpu.make_async_copy(bufs[p], dst, ssems[p]).wait()

  def add_pos(k, p):
    buf = bufs[p]
    phase2 = lax.rem(k * (CHUNK // 2), HL)

    @plsc.parallel_loop(0, CHUNK // 16, unroll=2)
    def rows(g):
      for u in range(16):
        r = g * 16 + u
        pr = phase2 + g * 8 + u // 2
        for v in range(D // 16):
          so = pl.ds(v * 16, 16)
          po = pl.ds((u % 2) * D + v * 16, 16)
          buf[r, so] = buf[r, so] + pos_v[pr, po]

  start_gather(0, 0)
  start_gather(1, 1)

  start_gather(2, 2)
  wait_gather(0)
  add_pos(0, 0)
  start_store(0, 0)

  def group(g, carry):
    for p_off in range(NBUF):
      k = NBUF * g + 1 + p_off
      pc = (1 + p_off) % NBUF
      pn = (p_off) % NBUF
      wait_store(k - 1, pn)
      start_gather(k + 2, pn)
      wait_gather(pc)
      add_pos(k, pc)
      start_store(k, pc)
    return carry

  lax.fori_loop(0, (N_CHUNKS - 5) // NBUF, group, 0)

  for k in range(N_CHUNKS - 4, N_CHUNKS):
    pc = k % NBUF
    pn = (k - 1) % NBUF
    wait_store(k - 1, pn)
    if k + 2 < N_CHUNKS:
      start_gather(k + 2, pn)
    wait_gather(pc)
    add_pos(k, pc)
    start_store(k, pc)
  wait_store(N_CHUNKS - 1, (N_CHUNKS - 1) % NBUF)


@jax.jit
def _run(idx, tok2, pos2):
  mesh = plsc.VectorSubcoreMesh(
      core_axis_name="c", subcore_axis_name="s", num_cores=NC,
      num_subcores=NS)
  f = pl.kernel(
      _body,
      out_type=jax.ShapeDtypeStruct((TOTAL, 2 * D), jnp.float32),
      mesh=mesh,
      scratch_types=[
          pltpu.VMEM((N_CHUNKS, CHUNK), jnp.int32),
          pltpu.VMEM((POS_EXT, 2 * D), jnp.float32),
          pltpu.VMEM((CHUNK, 2 * D), jnp.float32),
          pltpu.VMEM((CHUNK, 2 * D), jnp.float32),
          pltpu.VMEM((CHUNK, 2 * D), jnp.float32),
          pltpu.VMEM((CHUNK,), jnp.int32),
          pltpu.VMEM((CHUNK,), jnp.int32),
          pltpu.VMEM((CHUNK,), jnp.int32),
          pltpu.SemaphoreType.DMA,
          pltpu.SemaphoreType.DMA,
          pltpu.SemaphoreType.DMA,
          pltpu.SemaphoreType.DMA,
          pltpu.SemaphoreType.DMA,
          pltpu.SemaphoreType.DMA,
      ],
  )
  return f(idx, tok2, pos2)


def kernel(inputs, token_table, position_table):
  idx = inputs.astype(jnp.int32).reshape(TOTAL // CHUNK, CHUNK)
  tok2 = jnp.pad(token_table, ((0, 0), (0, D)))
  pos2 = position_table.reshape(HL, 2 * D)
  out2 = _run(idx, tok2, pos2)
  return out2[:, :D].reshape(B, L, D)
